# Initial kernel scaffold; baseline (speedup 1.0000x reference)
import jax
import jax.numpy as jnp
from jax.experimental import pallas as pl


def kernel(input_ids, wte, wpe):
    raise NotImplementedError("write your pallas kernel here")


# speedup vs baseline: 1.3342x; 1.3342x over previous
import functools

import jax
import jax.numpy as jnp
from jax import lax
from jax.experimental import pallas as pl
from jax.experimental.pallas import tpu as pltpu
from jax.experimental.pallas import tpu_sc as plsc

SEQ_LEN = 2048
N_EMBD = 768
NUM_CORES = 2
NUM_SUBCORES = 16
NUM_WORKERS = NUM_CORES * NUM_SUBCORES
ROWS_PER_WORKER = SEQ_LEN // NUM_WORKERS
LANES = 16
VECS_PER_ROW = N_EMBD // LANES


def _emb_body(ids_hbm, wte_hbm, wpe_hbm, out_hbm, idx_v, rows_v, wpe_v,
              gat_sem, lin_sem):
    wid = lax.axis_index("s") * NUM_CORES + lax.axis_index("c")
    base = wid * ROWS_PER_WORKER

    pltpu.sync_copy(ids_hbm.at[pl.ds(base, ROWS_PER_WORKER)], idx_v)

    gat = pltpu.async_copy(wte_hbm.at[idx_v], rows_v, gat_sem)
    lin = pltpu.async_copy(wpe_hbm.at[pl.ds(base, ROWS_PER_WORKER)], wpe_v,
                           lin_sem)
    gat.wait()
    lin.wait()

    def add_row(j, carry):
        for i in range(VECS_PER_ROW):
            sl = pl.ds(i * LANES, LANES)
            rows_v[j, sl] += wpe_v[j, sl]
        return carry

    lax.fori_loop(0, ROWS_PER_WORKER, add_row, 0, unroll=False)

    pltpu.sync_copy(rows_v, out_hbm.at[pl.ds(base, ROWS_PER_WORKER)])


@jax.jit
def _embedding(input_ids, wte, wpe):
    mesh = plsc.VectorSubcoreMesh(core_axis_name="c", subcore_axis_name="s")
    run = pl.kernel(
        _emb_body,
        out_type=jax.ShapeDtypeStruct((SEQ_LEN, N_EMBD), jnp.float32),
        mesh=mesh,
        scratch_types=[
            pltpu.VMEM((ROWS_PER_WORKER,), jnp.int32),
            pltpu.VMEM((ROWS_PER_WORKER, N_EMBD), jnp.float32),
            pltpu.VMEM((ROWS_PER_WORKER, N_EMBD), jnp.float32),
            pltpu.SemaphoreType.DMA,
            pltpu.SemaphoreType.DMA,
        ],
    )
    return run(input_ids, wte, wpe)


def kernel(input_ids, wte, wpe):
    out = _embedding(input_ids.astype(jnp.int32), wte, wpe)
    return out[None, :, :]
